# baseline (speedup 1.0000x reference)
import jax
import jax.numpy as jnp
from jax.experimental import pallas as pl

NUM_GROUP = 512
GROUP_SIZE = 32


def _fps(xyz, npoint):
    B, N, _ = xyz.shape

    def single(points):
        def body(i, state):
            dist, idxs, farthest = state
            centroid = points[farthest]
            d = jnp.sum((points - centroid) ** 2, axis=-1)
            dist = jnp.minimum(dist, d)
            idxs = idxs.at[i].set(farthest)
            farthest = jnp.argmax(dist).astype(jnp.int32)
            return (dist, idxs, farthest)

        dist0 = jnp.full((N,), 1e10, dtype=points.dtype)
        idxs0 = jnp.zeros((npoint,), dtype=jnp.int32)
        _, idxs, _ = jax.lax.fori_loop(0, npoint, body, (dist0, idxs0, jnp.int32(0)))
        return idxs

    fps_idx = jax.vmap(single)(xyz)
    fps_data = jnp.take_along_axis(xyz, fps_idx[..., None], axis=1)
    return fps_data, fps_idx


def _knn(xyz, centers, k):
    diff = xyz[:, :, None, :] - centers[:, None, :, :]
    distances = jnp.linalg.norm(diff, axis=-1)
    neg = -jnp.transpose(distances, (0, 2, 1))
    _, idx = jax.lax.top_k(neg, k)
    return idx


def _sub_kernel(nb_ref, c_ref, o_ref):
    o_ref[...] = nb_ref[...] - c_ref[...]


def kernel(xyz):
    B, N, _ = xyz.shape
    center, center_idx = _fps(xyz, NUM_GROUP)
    ori_idx = _knn(xyz, center, GROUP_SIZE)
    idx = ori_idx[-1]
    neighborhood = xyz.reshape(B * N, 3)[idx]
    cb = jnp.broadcast_to(center[0][:, None, :], (NUM_GROUP, GROUP_SIZE, 3))
    out = pl.pallas_call(
        _sub_kernel,
        out_shape=jax.ShapeDtypeStruct((NUM_GROUP, GROUP_SIZE, 3), jnp.float32),
    )(neighborhood, cb)
    return out[None], center, ori_idx, center_idx


# speedup vs baseline: 11.5679x; 11.5679x over previous
import functools

import jax
import jax.numpy as jnp
from jax import lax
from jax.experimental import pallas as pl
from jax.experimental.pallas import tpu as pltpu
from jax.experimental.pallas import tpu_sc as plsc

G = 512
S = 32
N = 16384
NC, NS, L = 2, 16, 16
NW = NC * NS
CPW = G // NW
CAP = 1024
IBIG = 0x3FFFFFFF


def _fps_body(x_ref, y_ref, z_ref, cidx_ref, cxyz_ref):
    X = x_ref[...]
    Y = y_ref[...]
    Z = z_ref[...]
    rows = lax.broadcasted_iota(jnp.int32, (128, 128), 0)
    cols = lax.broadcasted_iota(jnp.int32, (128, 128), 1)
    fi = rows * 128 + cols

    def body(i, carry):
        dist, far = carry
        m = fi == far
        cx = jnp.sum(jnp.where(m, X, 0.0))
        cy = jnp.sum(jnp.where(m, Y, 0.0))
        cz = jnp.sum(jnp.where(m, Z, 0.0))
        cidx_ref[i] = far
        cxyz_ref[i, 0] = cx
        cxyz_ref[i, 1] = cy
        cxyz_ref[i, 2] = cz
        dx = X - cx
        dy = Y - cy
        dz = Z - cz
        d = (dx * dx + dy * dy) + dz * dz
        dist = jnp.minimum(dist, d)
        mx = jnp.max(dist)
        far2 = jnp.min(jnp.where(dist == mx, fi, jnp.int32(2**31 - 1)))
        return dist, far2

    dist0 = jnp.full((128, 128), 1e10, dtype=jnp.float32)
    lax.fori_loop(0, G, body, (dist0, jnp.int32(0)))


def _fps_call(x2, y2, z2):
    return pl.pallas_call(
        _fps_body,
        out_shape=[
            jax.ShapeDtypeStruct((G,), jnp.int32),
            jax.ShapeDtypeStruct((G, 3), jnp.float32),
        ],
        out_specs=[
            pl.BlockSpec(memory_space=pltpu.SMEM),
            pl.BlockSpec(memory_space=pltpu.SMEM),
        ],
    )(x2, y2, z2)


_CB = 8


def _dist_body(cxyz_ref, x_ref, y_ref, z_ref, d_ref):
    i = pl.program_id(0)
    cb = cxyz_ref[pl.ds(i * _CB, _CB), :]
    cx = cb[:, 0:1]
    cy = cb[:, 1:2]
    cz = cb[:, 2:3]
    X = jnp.broadcast_to(x_ref[...], (_CB, N))
    Y = jnp.broadcast_to(y_ref[...], (_CB, N))
    Z = jnp.broadcast_to(z_ref[...], (_CB, N))
    dx = X - cx
    dy = Y - cy
    dz = Z - cz
    d = (dx * dx + dy * dy) + dz * dz
    d_ref[...] = jnp.sqrt(d)


def _dist_call(cxyz, xr, yr, zr):
    return pl.pallas_call(
        _dist_body,
        grid=(G // _CB,),
        in_specs=[
            pl.BlockSpec((G, 3), lambda i: (0, 0)),
            pl.BlockSpec((1, N), lambda i: (0, 0)),
            pl.BlockSpec((1, N), lambda i: (0, 0)),
            pl.BlockSpec((1, N), lambda i: (0, 0)),
        ],
        out_specs=pl.BlockSpec((_CB, N), lambda i: (i, 0)),
        out_shape=jax.ShapeDtypeStruct((G, N), jnp.float32),
    )(cxyz, xr, yr, zr)


@functools.cache
def _make_topk_call():
    sc_mesh = plsc.VectorSubcoreMesh(
        core_axis_name="c", subcore_axis_name="s", num_cores=NC, num_subcores=NS
    )
    return functools.partial(
        pl.kernel,
        out_type=[
            jax.ShapeDtypeStruct((G, S), jnp.int32),
            jax.ShapeDtypeStruct((G, 3 * S), jnp.float32),
        ],
        mesh=sc_mesh,
        compiler_params=pltpu.CompilerParams(needs_layout_passes=False),
        scratch_types=[
        pltpu.VMEM((N,), jnp.float32),
        pltpu.VMEM((N,), jnp.float32),
        pltpu.VMEM((N,), jnp.float32),
        pltpu.VMEM((N,), jnp.float32),
        pltpu.VMEM((CAP,), jnp.float32),
        pltpu.VMEM((CAP,), jnp.int32),
        pltpu.VMEM((L,), jnp.float32),
        pltpu.VMEM((L,), jnp.float32),
        pltpu.VMEM((L,), jnp.float32),
        pltpu.VMEM((CPW,), jnp.int32),
        pltpu.VMEM((S,), jnp.int32),
        pltpu.VMEM((3 * S,), jnp.float32),
        ],
    )(_topk_body)


def _topk_body(d_hbm, x_hbm, y_hbm, z_hbm, cidx_hbm, oi_hbm, nb_hbm,
               xv, yv, zv, drow, cvals, cinds, mycx, mycy, mycz, mycid,
               oist, nbst):
    cid = lax.axis_index("c")
    sid = lax.axis_index("s")
    wid = sid * NC + cid
    pltpu.sync_copy(x_hbm, xv)
    pltpu.sync_copy(y_hbm, yv)
    pltpu.sync_copy(z_hbm, zv)
    pltpu.sync_copy(cidx_hbm.at[pl.ds(wid * CPW, CPW)], mycid)
    cptv = mycid[...]
    mycx[...] = plsc.load_gather(xv, [cptv])
    mycy[...] = plsc.load_gather(yv, [cptv])
    mycz[...] = plsc.load_gather(zv, [cptv])

    lane = lax.broadcasted_iota(jnp.int32, (L,), 0)
    inf16 = jnp.full((L,), jnp.inf, jnp.float32)
    big16 = jnp.full((L,), IBIG, jnp.int32)
    lane0 = lane == 0

    def center_body(t, _):
        center = wid * CPW + t
        pltpu.sync_copy(d_hbm.at[center], drow)

        def p1(j, accs):
            a0, a1, a2, a3 = accs
            b = j * (4 * L)
            a0 = jnp.minimum(a0, drow[pl.ds(b, L)])
            a1 = jnp.minimum(a1, drow[pl.ds(b + L, L)])
            a2 = jnp.minimum(a2, drow[pl.ds(b + 2 * L, L)])
            a3 = jnp.minimum(a3, drow[pl.ds(b + 3 * L, L)])
            return a0, a1, a2, a3

        a0, a1, a2, a3 = lax.fori_loop(
            0, N // (4 * L), p1, (inf16, inf16, inf16, inf16))
        s0 = jnp.sort(a0)
        s1 = jnp.sort(a1)
        s2 = jnp.sort(a2)
        s3 = jnp.sort(a3)
        m = jnp.maximum(jnp.maximum(s0, s1), jnp.maximum(s2, s3))
        tau = jnp.max(jnp.where(lane <= 7, m, -jnp.inf))
        taub = jnp.full((L,), tau)

        def initb(j, c):
            cvals[pl.ds(j * L, L)] = inf16
            cinds[pl.ds(j * L, L)] = big16
            return c

        lax.fori_loop(0, CAP // L, initb, jnp.int32(0))

        def p2(j, cnt):
            b = j * L
            v = drow[pl.ds(b, L)]
            msk = v <= taub

            @pl.when(jnp.any(msk))
            def _():
                cc = jnp.minimum(cnt, CAP - L)
                plsc.store_compressed(cvals.at[pl.ds(cc, L)], v, mask=msk)
                plsc.store_compressed(cinds.at[pl.ds(cc, L)], lane + b,
                                      mask=msk)

            return cnt + jnp.sum(msk.astype(jnp.int32))

        cnt = lax.fori_loop(0, N // L, p2, jnp.int32(0))
        cnt = jnp.minimum(cnt, jnp.int32(CAP))
        nch = (cnt + (L - 1)) // L

        def ext(j, st):
            reslo, reshi = st

            def scan_c(jc, s2_):
                bv, bi, bp = s2_
                v = cvals[pl.ds(jc * L, L)]
                ii = cinds[pl.ds(jc * L, L)]
                pp = lane + jc * L
                take = (v < bv) | ((v == bv) & (ii < bi))
                bv = jnp.where(take, v, bv)
                bi = jnp.where(take, ii, bi)
                bp = jnp.where(take, pp, bp)
                return bv, bi, bp

            bv, bi, bp = lax.fori_loop(0, nch, scan_c, (inf16, big16, big16))
            mv = jnp.min(bv)
            mi = jnp.min(jnp.where(bv == mv, bi, big16))
            mp = jnp.min(jnp.where((bv == mv) & (bi == mi), bp, big16))
            plsc.store_scatter(cvals, [jnp.full((L,), mp)], inf16, mask=lane0)
            miv = jnp.full((L,), mi)
            reslo = jnp.where(lane == j, miv, reslo)
            reshi = jnp.where(lane == (j - L), miv, reshi)
            return reslo, reshi

        reslo, reshi = lax.fori_loop(0, S, ext, (big16, big16))

        tb = jnp.full((L,), t)
        cxb = plsc.load_gather(mycx, [tb])
        cyb = plsc.load_gather(mycy, [tb])
        czb = plsc.load_gather(mycz, [tb])
        gxl = plsc.load_gather(xv, [reslo]) - cxb
        gxh = plsc.load_gather(xv, [reshi]) - cxb
        gyl = plsc.load_gather(yv, [reslo]) - cyb
        gyh = plsc.load_gather(yv, [reshi]) - cyb
        gzl = plsc.load_gather(zv, [reslo]) - czb
        gzh = plsc.load_gather(zv, [reshi]) - czb
        i3l = lane * 3
        i3h = (lane + L) * 3
        plsc.store_scatter(nbst, [i3l], gxl)
        plsc.store_scatter(nbst, [i3h], gxh)
        plsc.store_scatter(nbst, [i3l + 1], gyl)
        plsc.store_scatter(nbst, [i3h + 1], gyh)
        plsc.store_scatter(nbst, [i3l + 2], gzl)
        plsc.store_scatter(nbst, [i3h + 2], gzh)
        oist[pl.ds(0, L)] = reslo
        oist[pl.ds(L, L)] = reshi
        pltpu.sync_copy(oist, oi_hbm.at[center])
        pltpu.sync_copy(nbst, nb_hbm.at[center])
        return jnp.int32(0)

    lax.fori_loop(0, CPW, center_body, jnp.int32(0))


def kernel(xyz):
    x = xyz[0, :, 0]
    y = xyz[0, :, 1]
    z = xyz[0, :, 2]
    cidx, cxyz = _fps_call(
        x.reshape(128, 128), y.reshape(128, 128), z.reshape(128, 128))
    d = _dist_call(cxyz, x.reshape(1, N), y.reshape(1, N), z.reshape(1, N))
    oi, nb = _make_topk_call()(d, x, y, z, cidx)
    return (nb.reshape(1, G, S, 3), cxyz.reshape(1, G, 3),
            oi.reshape(1, G, S), cidx.reshape(1, G))


# speedup vs baseline: 14.9668x; 1.2938x over previous
import functools

import jax
import jax.numpy as jnp
from jax import lax
from jax.experimental import pallas as pl
from jax.experimental.pallas import tpu as pltpu
from jax.experimental.pallas import tpu_sc as plsc

G = 512
S = 32
N = 16384
NC, NS, L = 2, 16, 16
NW = NC * NS
CPW = G // NW
CAP = 1024
IBIG = 0x3FFFFFFF


def _fps_body(x_ref, y_ref, z_ref, cidx_ref, cxyz_ref):
    X = x_ref[...]
    Y = y_ref[...]
    Z = z_ref[...]
    rows = lax.broadcasted_iota(jnp.int32, (128, 128), 0)
    cols = lax.broadcasted_iota(jnp.int32, (128, 128), 1)
    fi = rows * 128 + cols

    def body(i, carry):
        dist, far = carry
        m = fi == far
        cx = jnp.sum(jnp.where(m, X, 0.0))
        cy = jnp.sum(jnp.where(m, Y, 0.0))
        cz = jnp.sum(jnp.where(m, Z, 0.0))
        cidx_ref[i] = far
        cxyz_ref[i, 0] = cx
        cxyz_ref[i, 1] = cy
        cxyz_ref[i, 2] = cz
        dx = X - cx
        dy = Y - cy
        dz = Z - cz
        d = (dx * dx + dy * dy) + dz * dz
        dist = jnp.minimum(dist, d)
        mx = jnp.max(dist)
        far2 = jnp.min(jnp.where(dist == mx, fi, jnp.int32(2**31 - 1)))
        return dist, far2

    dist0 = jnp.full((128, 128), 1e10, dtype=jnp.float32)
    lax.fori_loop(0, G, body, (dist0, jnp.int32(0)))


def _fps_call(x2, y2, z2):
    return pl.pallas_call(
        _fps_body,
        out_shape=[
            jax.ShapeDtypeStruct((G,), jnp.int32),
            jax.ShapeDtypeStruct((G, 3), jnp.float32),
        ],
        out_specs=[
            pl.BlockSpec(memory_space=pltpu.SMEM),
            pl.BlockSpec(memory_space=pltpu.SMEM),
        ],
    )(x2, y2, z2)


_CB = 8


def _dist_body(cxyz_ref, x_ref, y_ref, z_ref, d_ref):
    i = pl.program_id(0)
    cb = cxyz_ref[pl.ds(i * _CB, _CB), :]
    cx = cb[:, 0:1]
    cy = cb[:, 1:2]
    cz = cb[:, 2:3]
    X = jnp.broadcast_to(x_ref[...], (_CB, N))
    Y = jnp.broadcast_to(y_ref[...], (_CB, N))
    Z = jnp.broadcast_to(z_ref[...], (_CB, N))
    dx = X - cx
    dy = Y - cy
    dz = Z - cz
    d = (dx * dx + dy * dy) + dz * dz
    d_ref[...] = jnp.sqrt(d)


def _dist_call(cxyz, xr, yr, zr):
    return pl.pallas_call(
        _dist_body,
        grid=(G // _CB,),
        in_specs=[
            pl.BlockSpec((G, 3), lambda i: (0, 0)),
            pl.BlockSpec((1, N), lambda i: (0, 0)),
            pl.BlockSpec((1, N), lambda i: (0, 0)),
            pl.BlockSpec((1, N), lambda i: (0, 0)),
        ],
        out_specs=pl.BlockSpec((_CB, N), lambda i: (i, 0)),
        out_shape=jax.ShapeDtypeStruct((G, N), jnp.float32),
    )(cxyz, xr, yr, zr)


@functools.cache
def _make_topk_call():
    sc_mesh = plsc.VectorSubcoreMesh(
        core_axis_name="c", subcore_axis_name="s", num_cores=NC, num_subcores=NS
    )
    return functools.partial(
        pl.kernel,
        out_type=[
            jax.ShapeDtypeStruct((G, S), jnp.int32),
            jax.ShapeDtypeStruct((G, 3 * S), jnp.float32),
        ],
        mesh=sc_mesh,
        compiler_params=pltpu.CompilerParams(needs_layout_passes=False),
        scratch_types=[
        pltpu.VMEM((N,), jnp.float32),
        pltpu.VMEM((N,), jnp.float32),
        pltpu.VMEM((N,), jnp.float32),
        pltpu.VMEM((N,), jnp.float32),
        pltpu.VMEM((CAP,), jnp.float32),
        pltpu.VMEM((CAP,), jnp.int32),
        pltpu.VMEM((L,), jnp.float32),
        pltpu.VMEM((L,), jnp.float32),
        pltpu.VMEM((L,), jnp.float32),
        pltpu.VMEM((CPW,), jnp.int32),
        pltpu.VMEM((S,), jnp.int32),
        pltpu.VMEM((3 * S,), jnp.float32),
        ],
    )(_topk_body)


def _topk_body(d_hbm, x_hbm, y_hbm, z_hbm, cidx_hbm, oi_hbm, nb_hbm,
               xv, yv, zv, drow, cvals, cinds, mycx, mycy, mycz, mycid,
               oist, nbst):
    cid = lax.axis_index("c")
    sid = lax.axis_index("s")
    wid = sid * NC + cid
    pltpu.sync_copy(x_hbm, xv)
    pltpu.sync_copy(y_hbm, yv)
    pltpu.sync_copy(z_hbm, zv)
    pltpu.sync_copy(cidx_hbm.at[pl.ds(wid * CPW, CPW)], mycid)
    cptv = mycid[...]
    mycx[...] = plsc.load_gather(xv, [cptv])
    mycy[...] = plsc.load_gather(yv, [cptv])
    mycz[...] = plsc.load_gather(zv, [cptv])

    lane = lax.broadcasted_iota(jnp.int32, (L,), 0)
    inf16 = jnp.full((L,), jnp.inf, jnp.float32)
    big16 = jnp.full((L,), IBIG, jnp.int32)
    lane0 = lane == 0

    def center_body(t, _):
        center = wid * CPW + t
        pltpu.sync_copy(d_hbm.at[center], drow)

        def p1(j, accs):
            a0, a1, a2, a3 = accs
            b = j * (4 * L)
            a0 = jnp.minimum(a0, drow[pl.ds(b, L)])
            a1 = jnp.minimum(a1, drow[pl.ds(b + L, L)])
            a2 = jnp.minimum(a2, drow[pl.ds(b + 2 * L, L)])
            a3 = jnp.minimum(a3, drow[pl.ds(b + 3 * L, L)])
            return a0, a1, a2, a3

        a0, a1, a2, a3 = lax.fori_loop(
            0, N // (4 * L), p1, (inf16, inf16, inf16, inf16))
        s0 = jnp.sort(a0)
        s1 = jnp.sort(a1)
        s2 = jnp.sort(a2)
        s3 = jnp.sort(a3)
        p01 = jnp.minimum(s0, lax.rev(s1, (0,)))
        p23 = jnp.minimum(s2, lax.rev(s3, (0,)))
        tau = jnp.max(jnp.maximum(p01, p23))
        taub = jnp.full((L,), tau)

        def initb(j, c):
            cvals[pl.ds(j * L, L)] = inf16
            cinds[pl.ds(j * L, L)] = big16
            return c

        lax.fori_loop(0, CAP // L, initb, jnp.int32(0))

        def p2(j, cnt):
            b = j * (4 * L)
            vs = [drow[pl.ds(b + k * L, L)] for k in range(4)]
            ms = [v <= taub for v in vs]
            hit = jnp.any((ms[0] | ms[1]) | (ms[2] | ms[3]))

            def hitfn(c):
                for k in range(4):
                    cc = jnp.minimum(c, CAP - L)
                    plsc.store_compressed(cvals.at[pl.ds(cc, L)], vs[k],
                                          mask=ms[k])
                    plsc.store_compressed(cinds.at[pl.ds(cc, L)],
                                          lane + (b + k * L), mask=ms[k])
                    pc = plsc.all_reduce_population_count(ms[k])
                    c = c + jnp.max(pc)
                return c

            return lax.cond(hit, hitfn, lambda c: c, cnt)

        cnt = lax.fori_loop(0, N // (4 * L), p2, jnp.int32(0))
        cnt = jnp.minimum(cnt, jnp.int32(CAP))
        nch = (cnt + (L - 1)) // L

        def ext(j, st):
            reslo, reshi = st

            def scan_c(jc, s2_):
                bv, bi, bp = s2_
                v = cvals[pl.ds(jc * L, L)]
                ii = cinds[pl.ds(jc * L, L)]
                pp = lane + jc * L
                take = (v < bv) | ((v == bv) & (ii < bi))
                bv = jnp.where(take, v, bv)
                bi = jnp.where(take, ii, bi)
                bp = jnp.where(take, pp, bp)
                return bv, bi, bp

            bv, bi, bp = lax.fori_loop(0, nch, scan_c, (inf16, big16, big16))
            mv = jnp.min(bv)
            mi = jnp.min(jnp.where(bv == mv, bi, big16))
            mp = jnp.min(jnp.where((bv == mv) & (bi == mi), bp, big16))
            plsc.store_scatter(cvals, [jnp.full((L,), mp)], inf16, mask=lane0)
            miv = jnp.full((L,), mi)
            reslo = jnp.where(lane == j, miv, reslo)
            reshi = jnp.where(lane == (j - L), miv, reshi)
            return reslo, reshi

        reslo, reshi = lax.fori_loop(0, S, ext, (big16, big16))

        tb = jnp.full((L,), t)
        cxb = plsc.load_gather(mycx, [tb])
        cyb = plsc.load_gather(mycy, [tb])
        czb = plsc.load_gather(mycz, [tb])
        gxl = plsc.load_gather(xv, [reslo]) - cxb
        gxh = plsc.load_gather(xv, [reshi]) - cxb
        gyl = plsc.load_gather(yv, [reslo]) - cyb
        gyh = plsc.load_gather(yv, [reshi]) - cyb
        gzl = plsc.load_gather(zv, [reslo]) - czb
        gzh = plsc.load_gather(zv, [reshi]) - czb
        i3l = lane * 3
        i3h = (lane + L) * 3
        plsc.store_scatter(nbst, [i3l], gxl)
        plsc.store_scatter(nbst, [i3h], gxh)
        plsc.store_scatter(nbst, [i3l + 1], gyl)
        plsc.store_scatter(nbst, [i3h + 1], gyh)
        plsc.store_scatter(nbst, [i3l + 2], gzl)
        plsc.store_scatter(nbst, [i3h + 2], gzh)
        oist[pl.ds(0, L)] = reslo
        oist[pl.ds(L, L)] = reshi
        pltpu.sync_copy(oist, oi_hbm.at[center])
        pltpu.sync_copy(nbst, nb_hbm.at[center])
        return jnp.int32(0)

    lax.fori_loop(0, CPW, center_body, jnp.int32(0))


def kernel(xyz):
    x = xyz[0, :, 0]
    y = xyz[0, :, 1]
    z = xyz[0, :, 2]
    cidx, cxyz = _fps_call(
        x.reshape(128, 128), y.reshape(128, 128), z.reshape(128, 128))
    d = _dist_call(cxyz, x.reshape(1, N), y.reshape(1, N), z.reshape(1, N))
    oi, nb = _make_topk_call()(d, x, y, z, cidx)
    return (nb.reshape(1, G, S, 3), cxyz.reshape(1, G, 3),
            oi.reshape(1, G, S), cidx.reshape(1, G))


# speedup vs baseline: 19.1654x; 1.2805x over previous
import functools

import jax
import jax.numpy as jnp
from jax import lax
from jax.experimental import pallas as pl
from jax.experimental.pallas import tpu as pltpu
from jax.experimental.pallas import tpu_sc as plsc

G = 512
S = 32
N = 16384
NC, NS, L = 2, 16, 16
NW = NC * NS
CPW = G // NW
CAP = 1024
IBIG = 0x3FFFFFFF


def _fps_body(x_ref, y_ref, z_ref, cidx_ref, cxyz_ref, dist_ref):
    cols1 = lax.broadcasted_iota(jnp.int32, (1, 128), 1)
    rows128 = lax.broadcasted_iota(jnp.int32, (128, 1), 0)

    def body(i, carry):
        dist, rs, cv = carry
        cmask = cols1 == cv
        cxv = jnp.sum(jnp.where(cmask, x_ref[pl.ds(rs, 1), :], 0.0),
                      axis=(0, 1), keepdims=True)
        cyv = jnp.sum(jnp.where(cmask, y_ref[pl.ds(rs, 1), :], 0.0),
                      axis=(0, 1), keepdims=True)
        czv = jnp.sum(jnp.where(cmask, z_ref[pl.ds(rs, 1), :], 0.0),
                      axis=(0, 1), keepdims=True)
        rsv = jnp.full((1, 1), rs, jnp.int32)
        cidx_ref[pl.ds(i, 1), :] = rsv * 128 + cv
        cxyz_ref[pl.ds(i, 1), :] = jnp.concatenate([cxv, cyv, czv], axis=1)
        dx = x_ref[...] - cxv
        dy = y_ref[...] - cyv
        dz = z_ref[...] - czv
        d = (dx * dx + dy * dy) + dz * dz
        dist = jnp.minimum(dist, d)
        dist_ref[...] = dist
        rm = jnp.max(dist, axis=1, keepdims=True)
        mxv = jnp.max(rm, axis=(0, 1), keepdims=True)
        rv = jnp.min(jnp.where(rm == mxv, rows128, jnp.int32(2**31 - 1)),
                     axis=(0, 1), keepdims=True)
        rs2 = rv[0, 0]
        drow_ = dist_ref[pl.ds(rs2, 1), :]
        cv2 = jnp.min(jnp.where(drow_ == mxv, cols1, jnp.int32(2**31 - 1)),
                      axis=(0, 1), keepdims=True)
        return dist, rs2, cv2

    dist0 = jnp.full((128, 128), 1e10, dtype=jnp.float32)
    cv0 = jnp.zeros((1, 1), jnp.int32)
    lax.fori_loop(0, G, body, (dist0, jnp.int32(0), cv0))


def _fps_call(x2, y2, z2):
    return pl.pallas_call(
        _fps_body,
        out_shape=[
            jax.ShapeDtypeStruct((G, 1), jnp.int32),
            jax.ShapeDtypeStruct((G, 3), jnp.float32),
        ],
        scratch_shapes=[pltpu.VMEM((128, 128), jnp.float32)],
    )(x2, y2, z2)


_CB = 8


def _dist_body(cxyz_ref, x_ref, y_ref, z_ref, d_ref, m_ref):
    i = pl.program_id(0)
    cb = cxyz_ref[pl.ds(i * _CB, _CB), :]
    cx = cb[:, 0:1]
    cy = cb[:, 1:2]
    cz = cb[:, 2:3]
    X = jnp.broadcast_to(x_ref[...], (_CB, N))
    Y = jnp.broadcast_to(y_ref[...], (_CB, N))
    Z = jnp.broadcast_to(z_ref[...], (_CB, N))
    dx = X - cx
    dy = Y - cy
    dz = Z - cz
    d = (dx * dx + dy * dy) + dz * dz
    dsq = jnp.sqrt(d)
    d_ref[...] = dsq
    m = dsq[:, 0:128]
    for k in range(1, 128):
        m = jnp.minimum(m, dsq[:, 128 * k:128 * (k + 1)])
    m_ref[...] = m


def _dist_call(cxyz, xr, yr, zr):
    return pl.pallas_call(
        _dist_body,
        grid=(G // _CB,),
        in_specs=[
            pl.BlockSpec((G, 3), lambda i: (0, 0)),
            pl.BlockSpec((1, N), lambda i: (0, 0)),
            pl.BlockSpec((1, N), lambda i: (0, 0)),
            pl.BlockSpec((1, N), lambda i: (0, 0)),
        ],
        out_specs=[
            pl.BlockSpec((_CB, N), lambda i: (i, 0)),
            pl.BlockSpec((_CB, 128), lambda i: (i, 0)),
        ],
        out_shape=[
            jax.ShapeDtypeStruct((G, N), jnp.float32),
            jax.ShapeDtypeStruct((G, 128), jnp.float32),
        ],
    )(cxyz, xr, yr, zr)


@functools.cache
def _make_topk_call():
    sc_mesh = plsc.VectorSubcoreMesh(
        core_axis_name="c", subcore_axis_name="s", num_cores=NC, num_subcores=NS
    )
    return functools.partial(
        pl.kernel,
        out_type=[
            jax.ShapeDtypeStruct((G, S), jnp.int32),
            jax.ShapeDtypeStruct((G, 3 * S), jnp.float32),
        ],
        mesh=sc_mesh,
        compiler_params=pltpu.CompilerParams(needs_layout_passes=False),
        scratch_types=[
        pltpu.VMEM((N,), jnp.float32),
        pltpu.VMEM((N,), jnp.float32),
        pltpu.VMEM((N,), jnp.float32),
        pltpu.VMEM((N,), jnp.float32),
        pltpu.VMEM((N,), jnp.float32),
        pltpu.VMEM((128,), jnp.float32),
        pltpu.VMEM((CAP,), jnp.float32),
        pltpu.VMEM((CAP,), jnp.int32),
        pltpu.VMEM((L,), jnp.float32),
        pltpu.VMEM((L,), jnp.float32),
        pltpu.VMEM((L,), jnp.float32),
        pltpu.VMEM((CPW,), jnp.int32),
        pltpu.VMEM((S,), jnp.int32),
        pltpu.VMEM((3 * S,), jnp.float32),
        pltpu.SemaphoreType.DMA,
        pltpu.SemaphoreType.DMA,
        ],
    )(_topk_body)


def _topk_body(d_hbm, m_hbm, x_hbm, y_hbm, z_hbm, cidx_hbm, oi_hbm, nb_hbm,
               xv, yv, zv, drow0, drow1, mrow, cvals, cinds,
               mycx, mycy, mycz, mycid, oist, nbst, sem0, sem1):
    cid = lax.axis_index("c")
    sid = lax.axis_index("s")
    wid = sid * NC + cid
    pltpu.sync_copy(x_hbm, xv)
    pltpu.sync_copy(y_hbm, yv)
    pltpu.sync_copy(z_hbm, zv)
    pltpu.sync_copy(cidx_hbm.at[pl.ds(wid * CPW, CPW)], mycid)
    cptv = mycid[...]
    mycx[...] = plsc.load_gather(xv, [cptv])
    mycy[...] = plsc.load_gather(yv, [cptv])
    mycz[...] = plsc.load_gather(zv, [cptv])

    lane = lax.broadcasted_iota(jnp.int32, (L,), 0)
    inf16 = jnp.full((L,), jnp.inf, jnp.float32)
    big16 = jnp.full((L,), IBIG, jnp.int32)
    lane0 = lane == 0

    pltpu.async_copy(d_hbm.at[wid * CPW], drow0, sem0)

    def center_pair(u, _):
        for b in range(2):
            t = u * 2 + b
            center = wid * CPW + t
            drow = drow0 if b == 0 else drow1
            sem = sem0 if b == 0 else sem1
            nxt_drow = drow1 if b == 0 else drow0
            nxt_sem = sem1 if b == 0 else sem0
            pltpu.make_async_copy(d_hbm.at[center], drow, sem).wait()

            @pl.when(t + 1 < CPW)
            def _():
                pltpu.async_copy(d_hbm.at[center + 1], nxt_drow, nxt_sem)

            pltpu.sync_copy(m_hbm.at[center], mrow)
            a0 = jnp.minimum(mrow[pl.ds(0, L)], mrow[pl.ds(4 * L, L)])
            a1 = jnp.minimum(mrow[pl.ds(L, L)], mrow[pl.ds(5 * L, L)])
            a2 = jnp.minimum(mrow[pl.ds(2 * L, L)], mrow[pl.ds(6 * L, L)])
            a3 = jnp.minimum(mrow[pl.ds(3 * L, L)], mrow[pl.ds(7 * L, L)])
            s0 = jnp.sort(a0)
            s1 = jnp.sort(a1)
            s2 = jnp.sort(a2)
            s3 = jnp.sort(a3)
            p01 = jnp.minimum(s0, lax.rev(s1, (0,)))
            p23 = jnp.minimum(s2, lax.rev(s3, (0,)))
            tau = jnp.max(jnp.maximum(p01, p23))
            taub = jnp.full((L,), tau)

            def p2(j, cnt):
                base = j * (8 * L)
                vs = [drow[pl.ds(base + k * L, L)] for k in range(8)]
                ms = [v <= taub for v in vs]
                o01 = ms[0] | ms[1]
                o23 = ms[2] | ms[3]
                o45 = ms[4] | ms[5]
                o67 = ms[6] | ms[7]
                hit = jnp.any((o01 | o23) | (o45 | o67))

                def hitfn(c):
                    pcs = [plsc.all_reduce_population_count(m)[0]
                           for m in ms]
                    for k in range(8):
                        cc = jnp.minimum(c, CAP - L)
                        plsc.store_compressed(cvals.at[pl.ds(cc, L)], vs[k],
                                              mask=ms[k])
                        plsc.store_compressed(cinds.at[pl.ds(cc, L)],
                                              lane + (base + k * L),
                                              mask=ms[k])
                        c = c + pcs[k]
                    return c

                return lax.cond(hit, hitfn, lambda c: c, cnt)

            cnt = lax.fori_loop(0, N // (8 * L), p2, jnp.int32(0))
            cnt = jnp.minimum(cnt, jnp.int32(CAP))
            nch = (cnt + (L - 1)) // L
            tail = lane + (nch - 1) * L
            plsc.store_scatter(cvals, [tail], inf16, mask=tail >= cnt)

            def ext(j, st):
                reslo, reshi = st

                def scan_c(jc, s2_):
                    bv, bi, bp = s2_
                    v = cvals[pl.ds(jc * L, L)]
                    ii = cinds[pl.ds(jc * L, L)]
                    pp = lane + jc * L
                    take = (v < bv) | ((v == bv) & (ii < bi))
                    bv = jnp.where(take, v, bv)
                    bi = jnp.where(take, ii, bi)
                    bp = jnp.where(take, pp, bp)
                    return bv, bi, bp

                bv, bi, bp = lax.fori_loop(0, nch, scan_c,
                                           (inf16, big16, big16))
                mv = jnp.min(bv)
                mi = jnp.min(jnp.where(bv == mv, bi, big16))
                mp = jnp.min(jnp.where((bv == mv) & (bi == mi), bp, big16))
                plsc.store_scatter(cvals, [jnp.full((L,), mp)], inf16,
                                   mask=lane0)
                miv = jnp.full((L,), mi)
                reslo = jnp.where(lane == j, miv, reslo)
                reshi = jnp.where(lane == (j - L), miv, reshi)
                return reslo, reshi

            reslo, reshi = lax.fori_loop(0, S, ext, (big16, big16))

            tb = jnp.full((L,), t)
            cxb = plsc.load_gather(mycx, [tb])
            cyb = plsc.load_gather(mycy, [tb])
            czb = plsc.load_gather(mycz, [tb])
            gxl = plsc.load_gather(xv, [reslo]) - cxb
            gxh = plsc.load_gather(xv, [reshi]) - cxb
            gyl = plsc.load_gather(yv, [reslo]) - cyb
            gyh = plsc.load_gather(yv, [reshi]) - cyb
            gzl = plsc.load_gather(zv, [reslo]) - czb
            gzh = plsc.load_gather(zv, [reshi]) - czb
            i3l = lane * 3
            i3h = (lane + L) * 3
            plsc.store_scatter(nbst, [i3l], gxl)
            plsc.store_scatter(nbst, [i3h], gxh)
            plsc.store_scatter(nbst, [i3l + 1], gyl)
            plsc.store_scatter(nbst, [i3h + 1], gyh)
            plsc.store_scatter(nbst, [i3l + 2], gzl)
            plsc.store_scatter(nbst, [i3h + 2], gzh)
            oist[pl.ds(0, L)] = reslo
            oist[pl.ds(L, L)] = reshi
            pltpu.sync_copy(oist, oi_hbm.at[center])
            pltpu.sync_copy(nbst, nb_hbm.at[center])
        return jnp.int32(0)

    lax.fori_loop(0, CPW // 2, center_pair, jnp.int32(0))


def kernel(xyz):
    x = xyz[0, :, 0]
    y = xyz[0, :, 1]
    z = xyz[0, :, 2]
    cidx2, cxyz = _fps_call(
        x.reshape(128, 128), y.reshape(128, 128), z.reshape(128, 128))
    cidx = cidx2.reshape(G)
    d, dm = _dist_call(cxyz, x.reshape(1, N), y.reshape(1, N), z.reshape(1, N))
    oi, nb = _make_topk_call()(d, dm, x, y, z, cidx)
    return (nb.reshape(1, G, S, 3), cxyz.reshape(1, G, 3),
            oi.reshape(1, G, S), cidx.reshape(1, G))


# speedup vs baseline: 20.7923x; 1.0849x over previous
import functools

import jax
import jax.numpy as jnp
from jax import lax
from jax.experimental import pallas as pl
from jax.experimental.pallas import tpu as pltpu
from jax.experimental.pallas import tpu_sc as plsc

G = 512
S = 32
N = 16384
NC, NS, L = 2, 16, 16
NW = NC * NS
CPW = G // NW
CAP = 1024
IBIG = 0x3FFFFFFF


def _fps_body(x_ref, y_ref, z_ref, cidx_ref, cxyz_ref):
    cols1 = lax.broadcasted_iota(jnp.int32, (1, 128), 1).astype(jnp.float32)
    rows128 = lax.broadcasted_iota(jnp.int32, (128, 1), 0).astype(jnp.float32)

    def body(i, carry):
        dist, rs, cv = carry
        cmask = cols1 == cv
        cxv = jnp.sum(jnp.where(cmask, x_ref[pl.ds(rs, 1), :], 0.0),
                      axis=(0, 1), keepdims=True)
        cyv = jnp.sum(jnp.where(cmask, y_ref[pl.ds(rs, 1), :], 0.0),
                      axis=(0, 1), keepdims=True)
        czv = jnp.sum(jnp.where(cmask, z_ref[pl.ds(rs, 1), :], 0.0),
                      axis=(0, 1), keepdims=True)
        rsv = jnp.full((1, 1), rs, jnp.int32)
        cidx_ref[pl.ds(i, 1), :] = rsv * 128 + cv.astype(jnp.int32)
        cxyz_ref[pl.ds(i, 1), :] = jnp.concatenate([cxv, cyv, czv], axis=1)
        dx = x_ref[...] - cxv
        dy = y_ref[...] - cyv
        dz = z_ref[...] - czv
        d = (dx * dx + dy * dy) + dz * dz
        dist = jnp.minimum(dist, d)
        rm = jnp.max(dist, axis=1, keepdims=True)
        mxv = jnp.max(rm, axis=(0, 1), keepdims=True)
        rv = jnp.min(jnp.where(rm == mxv, rows128, jnp.float32(3e8)),
                     axis=(0, 1), keepdims=True)
        rs2 = rv[0, 0].astype(jnp.int32)
        m2 = (dist == mxv) & (rows128 == rv)
        cv2 = jnp.min(jnp.where(m2, cols1, jnp.float32(3e8)),
                      axis=(0, 1), keepdims=True)
        return dist, rs2, cv2

    dist0 = jnp.full((128, 128), 1e10, dtype=jnp.float32)
    cv0 = jnp.zeros((1, 1), jnp.float32)
    lax.fori_loop(0, G, body, (dist0, jnp.int32(0), cv0))


def _fps_call(x2, y2, z2):
    return pl.pallas_call(
        _fps_body,
        out_shape=[
            jax.ShapeDtypeStruct((G, 1), jnp.int32),
            jax.ShapeDtypeStruct((G, 3), jnp.float32),
        ],
    )(x2, y2, z2)


_CB = 8


def _dist_body(cxyz_ref, x_ref, y_ref, z_ref, d_ref, m_ref):
    i = pl.program_id(0)
    cb = cxyz_ref[pl.ds(i * _CB, _CB), :]
    cx = cb[:, 0:1]
    cy = cb[:, 1:2]
    cz = cb[:, 2:3]
    X = jnp.broadcast_to(x_ref[...], (_CB, N))
    Y = jnp.broadcast_to(y_ref[...], (_CB, N))
    Z = jnp.broadcast_to(z_ref[...], (_CB, N))
    dx = X - cx
    dy = Y - cy
    dz = Z - cz
    d = (dx * dx + dy * dy) + dz * dz
    dsq = jnp.sqrt(d)
    d_ref[...] = dsq
    m = dsq[:, 0:128]
    for k in range(1, 128):
        m = jnp.minimum(m, dsq[:, 128 * k:128 * (k + 1)])
    m_ref[...] = m


def _dist_call(cxyz, xr, yr, zr):
    return pl.pallas_call(
        _dist_body,
        grid=(G // _CB,),
        in_specs=[
            pl.BlockSpec((G, 3), lambda i: (0, 0)),
            pl.BlockSpec((1, N), lambda i: (0, 0)),
            pl.BlockSpec((1, N), lambda i: (0, 0)),
            pl.BlockSpec((1, N), lambda i: (0, 0)),
        ],
        out_specs=[
            pl.BlockSpec((_CB, N), lambda i: (i, 0)),
            pl.BlockSpec((_CB, 128), lambda i: (i, 0)),
        ],
        out_shape=[
            jax.ShapeDtypeStruct((G, N), jnp.float32),
            jax.ShapeDtypeStruct((G, 128), jnp.float32),
        ],
    )(cxyz, xr, yr, zr)


@functools.cache
def _make_topk_call():
    sc_mesh = plsc.VectorSubcoreMesh(
        core_axis_name="c", subcore_axis_name="s", num_cores=NC, num_subcores=NS
    )
    return functools.partial(
        pl.kernel,
        out_type=[
            jax.ShapeDtypeStruct((G, S), jnp.int32),
            jax.ShapeDtypeStruct((G, 3 * S), jnp.float32),
        ],
        mesh=sc_mesh,
        compiler_params=pltpu.CompilerParams(needs_layout_passes=False),
        scratch_types=[
        pltpu.VMEM((N,), jnp.float32),
        pltpu.VMEM((N,), jnp.float32),
        pltpu.VMEM((N,), jnp.float32),
        pltpu.VMEM((N,), jnp.float32),
        pltpu.VMEM((N,), jnp.float32),
        pltpu.VMEM((128,), jnp.float32),
        pltpu.VMEM((CAP,), jnp.float32),
        pltpu.VMEM((CAP,), jnp.int32),
        pltpu.VMEM((L,), jnp.float32),
        pltpu.VMEM((L,), jnp.float32),
        pltpu.VMEM((L,), jnp.float32),
        pltpu.VMEM((CPW,), jnp.int32),
        pltpu.VMEM((S,), jnp.int32),
        pltpu.VMEM((3 * S,), jnp.float32),
        pltpu.SemaphoreType.DMA,
        pltpu.SemaphoreType.DMA,
        ],
    )(_topk_body)


def _topk_body(d_hbm, m_hbm, x_hbm, y_hbm, z_hbm, cidx_hbm, oi_hbm, nb_hbm,
               xv, yv, zv, drow0, drow1, mrow, cvals, cinds,
               mycx, mycy, mycz, mycid, oist, nbst, sem0, sem1):
    cid = lax.axis_index("c")
    sid = lax.axis_index("s")
    wid = sid * NC + cid
    pltpu.sync_copy(x_hbm, xv)
    pltpu.sync_copy(y_hbm, yv)
    pltpu.sync_copy(z_hbm, zv)
    pltpu.sync_copy(cidx_hbm.at[pl.ds(wid * CPW, CPW)], mycid)
    cptv = mycid[...]
    mycx[...] = plsc.load_gather(xv, [cptv])
    mycy[...] = plsc.load_gather(yv, [cptv])
    mycz[...] = plsc.load_gather(zv, [cptv])

    lane = lax.broadcasted_iota(jnp.int32, (L,), 0)
    inf16 = jnp.full((L,), jnp.inf, jnp.float32)
    big16 = jnp.full((L,), IBIG, jnp.int32)
    lane0 = lane == 0

    pltpu.async_copy(d_hbm.at[wid * CPW], drow0, sem0)

    def center_pair(u, _):
        for b in range(2):
            t = u * 2 + b
            center = wid * CPW + t
            drow = drow0 if b == 0 else drow1
            sem = sem0 if b == 0 else sem1
            nxt_drow = drow1 if b == 0 else drow0
            nxt_sem = sem1 if b == 0 else sem0
            pltpu.make_async_copy(d_hbm.at[center], drow, sem).wait()

            @pl.when(t + 1 < CPW)
            def _():
                pltpu.async_copy(d_hbm.at[center + 1], nxt_drow, nxt_sem)

            pltpu.sync_copy(m_hbm.at[center], mrow)
            a0 = jnp.minimum(mrow[pl.ds(0, L)], mrow[pl.ds(4 * L, L)])
            a1 = jnp.minimum(mrow[pl.ds(L, L)], mrow[pl.ds(5 * L, L)])
            a2 = jnp.minimum(mrow[pl.ds(2 * L, L)], mrow[pl.ds(6 * L, L)])
            a3 = jnp.minimum(mrow[pl.ds(3 * L, L)], mrow[pl.ds(7 * L, L)])
            s0 = jnp.sort(a0)
            s1 = jnp.sort(a1)
            s2 = jnp.sort(a2)
            s3 = jnp.sort(a3)
            p01 = jnp.minimum(s0, lax.rev(s1, (0,)))
            p23 = jnp.minimum(s2, lax.rev(s3, (0,)))
            tau = jnp.max(jnp.maximum(p01, p23))
            taub = jnp.full((L,), tau)

            def p2(j, cnt):
                base = j * (8 * L)
                vs = [drow[pl.ds(base + k * L, L)] for k in range(8)]
                ms = [v <= taub for v in vs]
                o01 = ms[0] | ms[1]
                o23 = ms[2] | ms[3]
                o45 = ms[4] | ms[5]
                o67 = ms[6] | ms[7]
                hit = jnp.any((o01 | o23) | (o45 | o67))

                def hitfn(c):
                    pcs = [plsc.all_reduce_population_count(m)[0]
                           for m in ms]
                    for k in range(8):
                        cc = jnp.minimum(c, CAP - L)
                        plsc.store_compressed(cvals.at[pl.ds(cc, L)], vs[k],
                                              mask=ms[k])
                        plsc.store_compressed(cinds.at[pl.ds(cc, L)],
                                              lane + (base + k * L),
                                              mask=ms[k])
                        c = c + pcs[k]
                    return c

                return lax.cond(hit, hitfn, lambda c: c, cnt)

            cnt = lax.fori_loop(0, N // (8 * L), p2, jnp.int32(0))
            cnt = jnp.minimum(cnt, jnp.int32(CAP))
            nch = (cnt + (L - 1)) // L
            tail = lane + (nch - 1) * L
            plsc.store_scatter(cvals, [tail], inf16, mask=tail >= cnt)

            def ext(j, st):
                reslo, reshi = st

                def scan_c(jc, s2_):
                    bv, bi, bp = s2_
                    v = cvals[pl.ds(jc * L, L)]
                    ii = cinds[pl.ds(jc * L, L)]
                    pp = lane + jc * L
                    take = (v < bv) | ((v == bv) & (ii < bi))
                    bv = jnp.where(take, v, bv)
                    bi = jnp.where(take, ii, bi)
                    bp = jnp.where(take, pp, bp)
                    return bv, bi, bp

                bv, bi, bp = lax.fori_loop(0, nch, scan_c,
                                           (inf16, big16, big16))
                mv = jnp.min(bv)
                mi = jnp.min(jnp.where(bv == mv, bi, big16))
                mp = jnp.min(jnp.where((bv == mv) & (bi == mi), bp, big16))
                plsc.store_scatter(cvals, [jnp.full((L,), mp)], inf16,
                                   mask=lane0)
                miv = jnp.full((L,), mi)
                reslo = jnp.where(lane == j, miv, reslo)
                reshi = jnp.where(lane == (j - L), miv, reshi)
                return reslo, reshi

            reslo, reshi = lax.fori_loop(0, S, ext, (big16, big16))

            tb = jnp.full((L,), t)
            cxb = plsc.load_gather(mycx, [tb])
            cyb = plsc.load_gather(mycy, [tb])
            czb = plsc.load_gather(mycz, [tb])
            gxl = plsc.load_gather(xv, [reslo]) - cxb
            gxh = plsc.load_gather(xv, [reshi]) - cxb
            gyl = plsc.load_gather(yv, [reslo]) - cyb
            gyh = plsc.load_gather(yv, [reshi]) - cyb
            gzl = plsc.load_gather(zv, [reslo]) - czb
            gzh = plsc.load_gather(zv, [reshi]) - czb
            i3l = lane * 3
            i3h = (lane + L) * 3
            plsc.store_scatter(nbst, [i3l], gxl)
            plsc.store_scatter(nbst, [i3h], gxh)
            plsc.store_scatter(nbst, [i3l + 1], gyl)
            plsc.store_scatter(nbst, [i3h + 1], gyh)
            plsc.store_scatter(nbst, [i3l + 2], gzl)
            plsc.store_scatter(nbst, [i3h + 2], gzh)
            oist[pl.ds(0, L)] = reslo
            oist[pl.ds(L, L)] = reshi
            pltpu.sync_copy(oist, oi_hbm.at[center])
            pltpu.sync_copy(nbst, nb_hbm.at[center])
        return jnp.int32(0)

    lax.fori_loop(0, CPW // 2, center_pair, jnp.int32(0))


def kernel(xyz):
    x = xyz[0, :, 0]
    y = xyz[0, :, 1]
    z = xyz[0, :, 2]
    cidx2, cxyz = _fps_call(
        x.reshape(128, 128), y.reshape(128, 128), z.reshape(128, 128))
    cidx = cidx2.reshape(G)
    d, dm = _dist_call(cxyz, x.reshape(1, N), y.reshape(1, N), z.reshape(1, N))
    oi, nb = _make_topk_call()(d, dm, x, y, z, cidx)
    return (nb.reshape(1, G, S, 3), cxyz.reshape(1, G, 3),
            oi.reshape(1, G, S), cidx.reshape(1, G))


# speedup vs baseline: 20.9683x; 1.0085x over previous
import functools

import jax
import jax.numpy as jnp
from jax import lax
from jax.experimental import pallas as pl
from jax.experimental.pallas import tpu as pltpu
from jax.experimental.pallas import tpu_sc as plsc

G = 512
S = 32
N = 16384
NC, NS, L = 2, 16, 16
NW = NC * NS
CPW = G // NW
CAP = 1024
IBIG = 0x3FFFFFFF


def _fps_body(x_ref, y_ref, z_ref, cidx_ref, cxyz_ref):
    cols1 = lax.broadcasted_iota(jnp.int32, (1, 128), 1).astype(jnp.float32)
    rows128 = lax.broadcasted_iota(jnp.int32, (128, 1), 0).astype(jnp.float32)

    def body(i, carry):
        dist, rs, cv = carry
        cmask = cols1 == cv
        cxv = jnp.sum(jnp.where(cmask, x_ref[pl.ds(rs, 1), :], 0.0),
                      axis=(0, 1), keepdims=True)
        cyv = jnp.sum(jnp.where(cmask, y_ref[pl.ds(rs, 1), :], 0.0),
                      axis=(0, 1), keepdims=True)
        czv = jnp.sum(jnp.where(cmask, z_ref[pl.ds(rs, 1), :], 0.0),
                      axis=(0, 1), keepdims=True)
        rsv = jnp.full((1, 1), rs, jnp.int32)
        cidx_ref[pl.ds(i, 1), :] = rsv * 128 + cv.astype(jnp.int32)
        cxyz_ref[pl.ds(i, 1), :] = jnp.concatenate([cxv, cyv, czv], axis=1)
        dx = x_ref[...] - cxv
        dy = y_ref[...] - cyv
        dz = z_ref[...] - czv
        d = (dx * dx + dy * dy) + dz * dz
        dist = jnp.minimum(dist, d)
        rm = jnp.max(dist, axis=1, keepdims=True)
        mxv = jnp.max(rm, axis=(0, 1), keepdims=True)
        rv = jnp.min(jnp.where(rm == mxv, rows128, jnp.float32(3e8)),
                     axis=(0, 1), keepdims=True)
        rs2 = rv[0, 0].astype(jnp.int32)
        m2 = (dist == mxv) & (rows128 == rv)
        cv2 = jnp.min(jnp.where(m2, cols1, jnp.float32(3e8)),
                      axis=(0, 1), keepdims=True)
        return dist, rs2, cv2

    dist0 = jnp.full((128, 128), 1e10, dtype=jnp.float32)
    cv0 = jnp.zeros((1, 1), jnp.float32)
    lax.fori_loop(0, G, body, (dist0, jnp.int32(0), cv0))


def _fps_call(x2, y2, z2):
    return pl.pallas_call(
        _fps_body,
        out_shape=[
            jax.ShapeDtypeStruct((G, 1), jnp.int32),
            jax.ShapeDtypeStruct((G, 3), jnp.float32),
        ],
    )(x2, y2, z2)


_CB = 8


def _dist_body(cxyz_ref, x_ref, y_ref, z_ref, d_ref, m_ref):
    i = pl.program_id(0)
    cb = cxyz_ref[pl.ds(i * _CB, _CB), :]
    cx = cb[:, 0:1]
    cy = cb[:, 1:2]
    cz = cb[:, 2:3]
    X = jnp.broadcast_to(x_ref[...], (_CB, N))
    Y = jnp.broadcast_to(y_ref[...], (_CB, N))
    Z = jnp.broadcast_to(z_ref[...], (_CB, N))
    dx = X - cx
    dy = Y - cy
    dz = Z - cz
    d = (dx * dx + dy * dy) + dz * dz
    dsq = jnp.sqrt(d)
    d_ref[...] = dsq
    m = dsq[:, 0:128]
    for k in range(1, 128):
        m = jnp.minimum(m, dsq[:, 128 * k:128 * (k + 1)])
    m_ref[...] = m


def _dist_call(cxyz, xr, yr, zr):
    return pl.pallas_call(
        _dist_body,
        grid=(G // _CB,),
        in_specs=[
            pl.BlockSpec((G, 3), lambda i: (0, 0)),
            pl.BlockSpec((1, N), lambda i: (0, 0)),
            pl.BlockSpec((1, N), lambda i: (0, 0)),
            pl.BlockSpec((1, N), lambda i: (0, 0)),
        ],
        out_specs=[
            pl.BlockSpec((_CB, N), lambda i: (i, 0)),
            pl.BlockSpec((_CB, 128), lambda i: (i, 0)),
        ],
        out_shape=[
            jax.ShapeDtypeStruct((G, N), jnp.float32),
            jax.ShapeDtypeStruct((G, 128), jnp.float32),
        ],
    )(cxyz, xr, yr, zr)


@functools.cache
def _make_topk_call():
    sc_mesh = plsc.VectorSubcoreMesh(
        core_axis_name="c", subcore_axis_name="s", num_cores=NC, num_subcores=NS
    )
    return functools.partial(
        pl.kernel,
        out_type=[
            jax.ShapeDtypeStruct((G, S), jnp.int32),
            jax.ShapeDtypeStruct((G, 3 * S), jnp.float32),
        ],
        mesh=sc_mesh,
        compiler_params=pltpu.CompilerParams(needs_layout_passes=False),
        scratch_types=[
        pltpu.VMEM((N,), jnp.float32),
        pltpu.VMEM((N,), jnp.float32),
        pltpu.VMEM((N,), jnp.float32),
        pltpu.VMEM((N,), jnp.float32),
        pltpu.VMEM((N,), jnp.float32),
        pltpu.VMEM((128,), jnp.float32),
        pltpu.VMEM((CAP,), jnp.float32),
        pltpu.VMEM((CAP,), jnp.int32),
        pltpu.VMEM((L,), jnp.float32),
        pltpu.VMEM((L,), jnp.float32),
        pltpu.VMEM((L,), jnp.float32),
        pltpu.VMEM((CPW,), jnp.int32),
        pltpu.VMEM((S,), jnp.int32),
        pltpu.VMEM((S,), jnp.int32),
        pltpu.VMEM((3 * S,), jnp.float32),
        pltpu.VMEM((3 * S,), jnp.float32),
        pltpu.SemaphoreType.DMA,
        pltpu.SemaphoreType.DMA,
        pltpu.SemaphoreType.DMA,
        pltpu.SemaphoreType.DMA,
        pltpu.SemaphoreType.DMA,
        pltpu.SemaphoreType.DMA,
        ],
    )(_topk_body)


def _topk_body(d_hbm, m_hbm, x_hbm, y_hbm, z_hbm, cidx_hbm, oi_hbm, nb_hbm,
               xv, yv, zv, drow0, drow1, mrow, cvals, cinds,
               mycx, mycy, mycz, mycid, oist0, oist1, nbst0, nbst1,
               sem0, sem1, osem0, osem1, nsem0, nsem1):
    cid = lax.axis_index("c")
    sid = lax.axis_index("s")
    wid = sid * NC + cid
    cp1 = pltpu.async_copy(x_hbm, xv, sem0)
    cp2 = pltpu.async_copy(y_hbm, yv, sem1)
    cp3 = pltpu.async_copy(z_hbm, zv, osem0)
    cp4 = pltpu.async_copy(cidx_hbm.at[pl.ds(wid * CPW, CPW)], mycid, osem1)
    cp1.wait()
    cp2.wait()
    cp3.wait()
    cp4.wait()
    cptv = mycid[...]
    mycx[...] = plsc.load_gather(xv, [cptv])
    mycy[...] = plsc.load_gather(yv, [cptv])
    mycz[...] = plsc.load_gather(zv, [cptv])

    lane = lax.broadcasted_iota(jnp.int32, (L,), 0)
    inf16 = jnp.full((L,), jnp.inf, jnp.float32)
    big16 = jnp.full((L,), IBIG, jnp.int32)
    lane0 = lane == 0

    pltpu.async_copy(d_hbm.at[wid * CPW], drow0, sem0)

    def center_pair(u, _):
        for b in range(2):
            t = u * 2 + b
            center = wid * CPW + t
            drow = drow0 if b == 0 else drow1
            sem = sem0 if b == 0 else sem1
            nxt_drow = drow1 if b == 0 else drow0
            nxt_sem = sem1 if b == 0 else sem0
            oist = oist0 if b == 0 else oist1
            nbst = nbst0 if b == 0 else nbst1
            osem = osem0 if b == 0 else osem1
            nsem = nsem0 if b == 0 else nsem1
            pltpu.make_async_copy(d_hbm.at[center], drow, sem).wait()

            @pl.when(t >= 2)
            def _():
                pltpu.make_async_copy(oist, oi_hbm.at[center - 2],
                                      osem).wait()
                pltpu.make_async_copy(nbst, nb_hbm.at[center - 2],
                                      nsem).wait()

            @pl.when(t + 1 < CPW)
            def _():
                pltpu.async_copy(d_hbm.at[center + 1], nxt_drow, nxt_sem)

            pltpu.sync_copy(m_hbm.at[center], mrow)
            a0 = jnp.minimum(mrow[pl.ds(0, L)], mrow[pl.ds(4 * L, L)])
            a1 = jnp.minimum(mrow[pl.ds(L, L)], mrow[pl.ds(5 * L, L)])
            a2 = jnp.minimum(mrow[pl.ds(2 * L, L)], mrow[pl.ds(6 * L, L)])
            a3 = jnp.minimum(mrow[pl.ds(3 * L, L)], mrow[pl.ds(7 * L, L)])
            s0 = jnp.sort(a0)
            s1 = jnp.sort(a1)
            s2 = jnp.sort(a2)
            s3 = jnp.sort(a3)
            p01 = jnp.minimum(s0, lax.rev(s1, (0,)))
            p23 = jnp.minimum(s2, lax.rev(s3, (0,)))
            tau = jnp.max(jnp.maximum(p01, p23))
            taub = jnp.full((L,), tau)

            def p2(j, cnt):
                base = j * (8 * L)
                vs = [drow[pl.ds(base + k * L, L)] for k in range(8)]
                ms = [v <= taub for v in vs]
                o01 = ms[0] | ms[1]
                o23 = ms[2] | ms[3]
                o45 = ms[4] | ms[5]
                o67 = ms[6] | ms[7]
                hit = jnp.any((o01 | o23) | (o45 | o67))

                def hitfn(c):
                    pcs = [plsc.all_reduce_population_count(m)[0]
                           for m in ms]
                    for k in range(8):
                        cc = jnp.minimum(c, CAP - L)
                        plsc.store_compressed(cvals.at[pl.ds(cc, L)], vs[k],
                                              mask=ms[k])
                        plsc.store_compressed(cinds.at[pl.ds(cc, L)],
                                              lane + (base + k * L),
                                              mask=ms[k])
                        c = c + pcs[k]
                    return c

                return lax.cond(hit, hitfn, lambda c: c, cnt)

            cnt = lax.fori_loop(0, N // (8 * L), p2, jnp.int32(0))
            cnt = jnp.minimum(cnt, jnp.int32(CAP))
            nch = (cnt + (L - 1)) // L
            tail = lane + (nch - 1) * L
            plsc.store_scatter(cvals, [tail], inf16, mask=tail >= cnt)

            def ext(j, st):
                reslo, reshi = st

                def scan_c(jc, s2_):
                    bv, bi, bp = s2_
                    v = cvals[pl.ds(jc * L, L)]
                    ii = cinds[pl.ds(jc * L, L)]
                    pp = lane + jc * L
                    take = (v < bv) | ((v == bv) & (ii < bi))
                    bv = jnp.where(take, v, bv)
                    bi = jnp.where(take, ii, bi)
                    bp = jnp.where(take, pp, bp)
                    return bv, bi, bp

                bv, bi, bp = lax.fori_loop(0, nch, scan_c,
                                           (inf16, big16, big16))
                mv = jnp.min(bv)
                mi = jnp.min(jnp.where(bv == mv, bi, big16))
                mp = jnp.min(jnp.where((bv == mv) & (bi == mi), bp, big16))
                plsc.store_scatter(cvals, [jnp.full((L,), mp)], inf16,
                                   mask=lane0)
                miv = jnp.full((L,), mi)
                reslo = jnp.where(lane == j, miv, reslo)
                reshi = jnp.where(lane == (j - L), miv, reshi)
                return reslo, reshi

            reslo, reshi = lax.fori_loop(0, S, ext, (big16, big16))

            tb = jnp.full((L,), t)
            cxb = plsc.load_gather(mycx, [tb])
            cyb = plsc.load_gather(mycy, [tb])
            czb = plsc.load_gather(mycz, [tb])
            gxl = plsc.load_gather(xv, [reslo]) - cxb
            gxh = plsc.load_gather(xv, [reshi]) - cxb
            gyl = plsc.load_gather(yv, [reslo]) - cyb
            gyh = plsc.load_gather(yv, [reshi]) - cyb
            gzl = plsc.load_gather(zv, [reslo]) - czb
            gzh = plsc.load_gather(zv, [reshi]) - czb
            i3l = lane * 3
            i3h = (lane + L) * 3
            plsc.store_scatter(nbst, [i3l], gxl)
            plsc.store_scatter(nbst, [i3h], gxh)
            plsc.store_scatter(nbst, [i3l + 1], gyl)
            plsc.store_scatter(nbst, [i3h + 1], gyh)
            plsc.store_scatter(nbst, [i3l + 2], gzl)
            plsc.store_scatter(nbst, [i3h + 2], gzh)
            oist[pl.ds(0, L)] = reslo
            oist[pl.ds(L, L)] = reshi
            pltpu.async_copy(oist, oi_hbm.at[center], osem)
            pltpu.async_copy(nbst, nb_hbm.at[center], nsem)
        return jnp.int32(0)

    lax.fori_loop(0, CPW // 2, center_pair, jnp.int32(0))
    base = wid * CPW + CPW - 2
    pltpu.make_async_copy(oist0, oi_hbm.at[base], osem0).wait()
    pltpu.make_async_copy(nbst0, nb_hbm.at[base], nsem0).wait()
    pltpu.make_async_copy(oist1, oi_hbm.at[base + 1], osem1).wait()
    pltpu.make_async_copy(nbst1, nb_hbm.at[base + 1], nsem1).wait()


def kernel(xyz):
    x = xyz[0, :, 0]
    y = xyz[0, :, 1]
    z = xyz[0, :, 2]
    cidx2, cxyz = _fps_call(
        x.reshape(128, 128), y.reshape(128, 128), z.reshape(128, 128))
    cidx = cidx2.reshape(G)
    d, dm = _dist_call(cxyz, x.reshape(1, N), y.reshape(1, N), z.reshape(1, N))
    oi, nb = _make_topk_call()(d, dm, x, y, z, cidx)
    return (nb.reshape(1, G, S, 3), cxyz.reshape(1, G, 3),
            oi.reshape(1, G, S), cidx.reshape(1, G))


# speedup vs baseline: 21.0415x; 1.0035x over previous
import functools

import jax
import jax.numpy as jnp
from jax import lax
from jax.experimental import pallas as pl
from jax.experimental.pallas import tpu as pltpu
from jax.experimental.pallas import tpu_sc as plsc

G = 512
S = 32
N = 16384
NC, NS, L = 2, 16, 16
NW = NC * NS
CPW = G // NW
CAP = 1024
IBIG = 0x3FFFFFFF


def _fps_body(x_ref, y_ref, z_ref, cidx_ref, cxyz_ref):
    cols1 = lax.broadcasted_iota(jnp.int32, (1, 128), 1).astype(jnp.float32)
    rows128 = lax.broadcasted_iota(jnp.int32, (128, 1), 0).astype(jnp.float32)

    def body(i, carry):
        dist, rs, cv = carry
        cmask = cols1 == cv
        cxv = jnp.sum(jnp.where(cmask, x_ref[pl.ds(rs, 1), :], 0.0),
                      axis=(0, 1), keepdims=True)
        cyv = jnp.sum(jnp.where(cmask, y_ref[pl.ds(rs, 1), :], 0.0),
                      axis=(0, 1), keepdims=True)
        czv = jnp.sum(jnp.where(cmask, z_ref[pl.ds(rs, 1), :], 0.0),
                      axis=(0, 1), keepdims=True)
        rsv = jnp.full((1, 1), rs, jnp.int32)
        cidx_ref[pl.ds(i, 1), :] = rsv * 128 + cv.astype(jnp.int32)
        cxyz_ref[pl.ds(i, 1), :] = jnp.concatenate([cxv, cyv, czv], axis=1)
        dx = x_ref[...] - cxv
        dy = y_ref[...] - cyv
        dz = z_ref[...] - czv
        d = (dx * dx + dy * dy) + dz * dz
        dist = jnp.minimum(dist, d)
        rm = jnp.max(dist, axis=1, keepdims=True)
        mxv = jnp.max(rm, axis=(0, 1), keepdims=True)
        rv = jnp.min(jnp.where(rm == mxv, rows128, jnp.float32(3e8)),
                     axis=(0, 1), keepdims=True)
        rs2 = rv[0, 0].astype(jnp.int32)
        m2 = (dist == mxv) & (rows128 == rv)
        cv2 = jnp.min(jnp.where(m2, cols1, jnp.float32(3e8)),
                      axis=(0, 1), keepdims=True)
        return dist, rs2, cv2

    dist0 = jnp.full((128, 128), 1e10, dtype=jnp.float32)
    cv0 = jnp.zeros((1, 1), jnp.float32)
    lax.fori_loop(0, G, body, (dist0, jnp.int32(0), cv0))


_CB = 8


def _fused_body(x2_ref, y2_ref, z2_ref, x_ref, y_ref, z_ref,
                cidx_ref, cxyz_ref, d_ref, m_ref):
    i = pl.program_id(0)

    @pl.when(i == 0)
    def _():
        _fps_body(x2_ref, y2_ref, z2_ref, cidx_ref, cxyz_ref)

    _dist_block(cxyz_ref, x_ref, y_ref, z_ref, d_ref, i)
    m = d_ref[:, 0:128]
    for k in range(1, 128):
        m = jnp.minimum(m, d_ref[:, 128 * k:128 * (k + 1)])
    m_ref[...] = m


def _fused_call(x2, y2, z2, xr, yr, zr):
    return pl.pallas_call(
        _fused_body,
        grid=(G // _CB,),
        in_specs=[
            pl.BlockSpec((128, 128), lambda i: (0, 0)),
            pl.BlockSpec((128, 128), lambda i: (0, 0)),
            pl.BlockSpec((128, 128), lambda i: (0, 0)),
            pl.BlockSpec((1, N), lambda i: (0, 0)),
            pl.BlockSpec((1, N), lambda i: (0, 0)),
            pl.BlockSpec((1, N), lambda i: (0, 0)),
        ],
        out_specs=[
            pl.BlockSpec((G, 1), lambda i: (0, 0)),
            pl.BlockSpec((G, 3), lambda i: (0, 0)),
            pl.BlockSpec((_CB, N), lambda i: (i, 0)),
            pl.BlockSpec((_CB, 128), lambda i: (i, 0)),
        ],
        out_shape=[
            jax.ShapeDtypeStruct((G, 1), jnp.int32),
            jax.ShapeDtypeStruct((G, 3), jnp.float32),
            jax.ShapeDtypeStruct((G, N), jnp.float32),
            jax.ShapeDtypeStruct((G, 128), jnp.float32),
        ],
    )(x2, y2, z2, xr, yr, zr)


def _dist_block(cxyz_ref, x_ref, y_ref, z_ref, d_ref, i):
    cb = cxyz_ref[pl.ds(i * _CB, _CB), :]
    cx = cb[:, 0:1]
    cy = cb[:, 1:2]
    cz = cb[:, 2:3]
    X = jnp.broadcast_to(x_ref[...], (_CB, N))
    Y = jnp.broadcast_to(y_ref[...], (_CB, N))
    Z = jnp.broadcast_to(z_ref[...], (_CB, N))
    dx = X - cx
    dy = Y - cy
    dz = Z - cz
    d = (dx * dx + dy * dy) + dz * dz
    d_ref[...] = jnp.sqrt(d)


@functools.cache
def _make_topk_call():
    sc_mesh = plsc.VectorSubcoreMesh(
        core_axis_name="c", subcore_axis_name="s", num_cores=NC, num_subcores=NS
    )
    return functools.partial(
        pl.kernel,
        out_type=[
            jax.ShapeDtypeStruct((G, S), jnp.int32),
            jax.ShapeDtypeStruct((G, 3 * S), jnp.float32),
        ],
        mesh=sc_mesh,
        compiler_params=pltpu.CompilerParams(needs_layout_passes=False),
        scratch_types=[
        pltpu.VMEM((N,), jnp.float32),
        pltpu.VMEM((N,), jnp.float32),
        pltpu.VMEM((N,), jnp.float32),
        pltpu.VMEM((N,), jnp.float32),
        pltpu.VMEM((N,), jnp.float32),
        pltpu.VMEM((128,), jnp.float32),
        pltpu.VMEM((CAP,), jnp.float32),
        pltpu.VMEM((CAP,), jnp.int32),
        pltpu.VMEM((L,), jnp.float32),
        pltpu.VMEM((L,), jnp.float32),
        pltpu.VMEM((L,), jnp.float32),
        pltpu.VMEM((CPW,), jnp.int32),
        pltpu.VMEM((S,), jnp.int32),
        pltpu.VMEM((S,), jnp.int32),
        pltpu.VMEM((3 * S,), jnp.float32),
        pltpu.VMEM((3 * S,), jnp.float32),
        pltpu.SemaphoreType.DMA,
        pltpu.SemaphoreType.DMA,
        pltpu.SemaphoreType.DMA,
        pltpu.SemaphoreType.DMA,
        pltpu.SemaphoreType.DMA,
        pltpu.SemaphoreType.DMA,
        ],
    )(_topk_body)


def _topk_body(d_hbm, m_hbm, x_hbm, y_hbm, z_hbm, cidx_hbm, oi_hbm, nb_hbm,
               xv, yv, zv, drow0, drow1, mrow, cvals, cinds,
               mycx, mycy, mycz, mycid, oist0, oist1, nbst0, nbst1,
               sem0, sem1, osem0, osem1, nsem0, nsem1):
    cid = lax.axis_index("c")
    sid = lax.axis_index("s")
    wid = sid * NC + cid
    cp1 = pltpu.async_copy(x_hbm, xv, sem0)
    cp2 = pltpu.async_copy(y_hbm, yv, sem1)
    cp3 = pltpu.async_copy(z_hbm, zv, osem0)
    cp4 = pltpu.async_copy(cidx_hbm.at[pl.ds(wid * CPW, CPW)], mycid, osem1)
    cp1.wait()
    cp2.wait()
    cp3.wait()
    cp4.wait()
    cptv = mycid[...]
    mycx[...] = plsc.load_gather(xv, [cptv])
    mycy[...] = plsc.load_gather(yv, [cptv])
    mycz[...] = plsc.load_gather(zv, [cptv])

    lane = lax.broadcasted_iota(jnp.int32, (L,), 0)
    inf16 = jnp.full((L,), jnp.inf, jnp.float32)
    big16 = jnp.full((L,), IBIG, jnp.int32)
    lane0 = lane == 0

    pltpu.async_copy(d_hbm.at[wid * CPW], drow0, sem0)

    def center_pair(u, _):
        for b in range(2):
            t = u * 2 + b
            center = wid * CPW + t
            drow = drow0 if b == 0 else drow1
            sem = sem0 if b == 0 else sem1
            nxt_drow = drow1 if b == 0 else drow0
            nxt_sem = sem1 if b == 0 else sem0
            oist = oist0 if b == 0 else oist1
            nbst = nbst0 if b == 0 else nbst1
            osem = osem0 if b == 0 else osem1
            nsem = nsem0 if b == 0 else nsem1
            pltpu.make_async_copy(d_hbm.at[center], drow, sem).wait()

            @pl.when(t >= 2)
            def _():
                pltpu.make_async_copy(oist, oi_hbm.at[center - 2],
                                      osem).wait()
                pltpu.make_async_copy(nbst, nb_hbm.at[center - 2],
                                      nsem).wait()

            @pl.when(t + 1 < CPW)
            def _():
                pltpu.async_copy(d_hbm.at[center + 1], nxt_drow, nxt_sem)

            pltpu.sync_copy(m_hbm.at[center], mrow)
            a0 = jnp.minimum(mrow[pl.ds(0, L)], mrow[pl.ds(4 * L, L)])
            a1 = jnp.minimum(mrow[pl.ds(L, L)], mrow[pl.ds(5 * L, L)])
            a2 = jnp.minimum(mrow[pl.ds(2 * L, L)], mrow[pl.ds(6 * L, L)])
            a3 = jnp.minimum(mrow[pl.ds(3 * L, L)], mrow[pl.ds(7 * L, L)])
            s0 = jnp.sort(a0)
            s1 = jnp.sort(a1)
            s2 = jnp.sort(a2)
            s3 = jnp.sort(a3)
            p01 = jnp.minimum(s0, lax.rev(s1, (0,)))
            p23 = jnp.minimum(s2, lax.rev(s3, (0,)))
            tau = jnp.max(jnp.maximum(p01, p23))
            taub = jnp.full((L,), tau)

            def p2(j, cnt):
                base = j * (8 * L)
                vs = [drow[pl.ds(base + k * L, L)] for k in range(8)]
                ms = [v <= taub for v in vs]
                o01 = ms[0] | ms[1]
                o23 = ms[2] | ms[3]
                o45 = ms[4] | ms[5]
                o67 = ms[6] | ms[7]
                hit = jnp.any((o01 | o23) | (o45 | o67))

                def hitfn(c):
                    pcs = [plsc.all_reduce_population_count(m)[0]
                           for m in ms]
                    for k in range(8):
                        cc = jnp.minimum(c, CAP - L)
                        plsc.store_compressed(cvals.at[pl.ds(cc, L)], vs[k],
                                              mask=ms[k])
                        plsc.store_compressed(cinds.at[pl.ds(cc, L)],
                                              lane + (base + k * L),
                                              mask=ms[k])
                        c = c + pcs[k]
                    return c

                return lax.cond(hit, hitfn, lambda c: c, cnt)

            cnt = lax.fori_loop(0, N // (8 * L), p2, jnp.int32(0))
            cnt = jnp.minimum(cnt, jnp.int32(CAP))
            nch = (cnt + (L - 1)) // L
            tail = lane + (nch - 1) * L
            plsc.store_scatter(cvals, [tail], inf16, mask=tail >= cnt)

            def ext(j, st):
                reslo, reshi = st

                def scan_c(jc, s2_):
                    bv, bi, bp = s2_
                    v = cvals[pl.ds(jc * L, L)]
                    ii = cinds[pl.ds(jc * L, L)]
                    pp = lane + jc * L
                    take = (v < bv) | ((v == bv) & (ii < bi))
                    bv = jnp.where(take, v, bv)
                    bi = jnp.where(take, ii, bi)
                    bp = jnp.where(take, pp, bp)
                    return bv, bi, bp

                bv, bi, bp = lax.fori_loop(0, nch, scan_c,
                                           (inf16, big16, big16))
                mv = jnp.min(bv)
                mi = jnp.min(jnp.where(bv == mv, bi, big16))
                mp = jnp.min(jnp.where((bv == mv) & (bi == mi), bp, big16))
                plsc.store_scatter(cvals, [jnp.full((L,), mp)], inf16,
                                   mask=lane0)
                miv = jnp.full((L,), mi)
                reslo = jnp.where(lane == j, miv, reslo)
                reshi = jnp.where(lane == (j - L), miv, reshi)
                return reslo, reshi

            reslo, reshi = lax.fori_loop(0, S, ext, (big16, big16))

            tb = jnp.full((L,), t)
            cxb = plsc.load_gather(mycx, [tb])
            cyb = plsc.load_gather(mycy, [tb])
            czb = plsc.load_gather(mycz, [tb])
            gxl = plsc.load_gather(xv, [reslo]) - cxb
            gxh = plsc.load_gather(xv, [reshi]) - cxb
            gyl = plsc.load_gather(yv, [reslo]) - cyb
            gyh = plsc.load_gather(yv, [reshi]) - cyb
            gzl = plsc.load_gather(zv, [reslo]) - czb
            gzh = plsc.load_gather(zv, [reshi]) - czb
            i3l = lane * 3
            i3h = (lane + L) * 3
            plsc.store_scatter(nbst, [i3l], gxl)
            plsc.store_scatter(nbst, [i3h], gxh)
            plsc.store_scatter(nbst, [i3l + 1], gyl)
            plsc.store_scatter(nbst, [i3h + 1], gyh)
            plsc.store_scatter(nbst, [i3l + 2], gzl)
            plsc.store_scatter(nbst, [i3h + 2], gzh)
            oist[pl.ds(0, L)] = reslo
            oist[pl.ds(L, L)] = reshi
            pltpu.async_copy(oist, oi_hbm.at[center], osem)
            pltpu.async_copy(nbst, nb_hbm.at[center], nsem)
        return jnp.int32(0)

    lax.fori_loop(0, CPW // 2, center_pair, jnp.int32(0))
    base = wid * CPW + CPW - 2
    pltpu.make_async_copy(oist0, oi_hbm.at[base], osem0).wait()
    pltpu.make_async_copy(nbst0, nb_hbm.at[base], nsem0).wait()
    pltpu.make_async_copy(oist1, oi_hbm.at[base + 1], osem1).wait()
    pltpu.make_async_copy(nbst1, nb_hbm.at[base + 1], nsem1).wait()


def kernel(xyz):
    x = xyz[0, :, 0]
    y = xyz[0, :, 1]
    z = xyz[0, :, 2]
    cidx2, cxyz, d, dm = _fused_call(
        x.reshape(128, 128), y.reshape(128, 128), z.reshape(128, 128),
        x.reshape(1, N), y.reshape(1, N), z.reshape(1, N))
    cidx = cidx2.reshape(G)
    oi, nb = _make_topk_call()(d, dm, x, y, z, cidx)
    return (nb.reshape(1, G, S, 3), cxyz.reshape(1, G, 3),
            oi.reshape(1, G, S), cidx.reshape(1, G))
